# baseline (speedup 1.0000x reference)
import functools

import jax
import jax.numpy as jnp
from jax.experimental import pallas as pl
from jax.experimental.pallas import tpu as pltpu

N = 10000
E = 320000
D = 128
DE = 16
NC = 3
G = 100
EPS = 1e-5
BE = 3200
NEB = E // BE


def _softplus(x):
    return jnp.maximum(x, 0.0) + jnp.log1p(jnp.exp(-jnp.abs(x)))


def _sigmoid(x):
    return 1.0 / (1.0 + jnp.exp(-x))


def _edge_body(w1st_ref, w2st_ref, west_ref, t_ref, g1_ref, g2_ref, ef_ref,
               w_ref):
    z = jnp.dot(g1_ref[...], w1st_ref[...], preferred_element_type=jnp.float32)
    z += jnp.dot(g2_ref[...], w2st_ref[...], preferred_element_type=jnp.float32)
    z += jnp.dot(ef_ref[...], west_ref[...], preferred_element_type=jnp.float32)
    z += t_ref[...]
    gate = _sigmoid(z[:, :D])
    conv = _softplus(z[:, D:])
    w_ref[...] = gate * conv


def _edge_stream(w1st, w2st, west, t, g1, g2, ef):
    return pl.pallas_call(
        _edge_body,
        grid=(NEB,),
        in_specs=[
            pl.BlockSpec((D, 2 * D), lambda i: (0, 0)),
            pl.BlockSpec((D, 2 * D), lambda i: (0, 0)),
            pl.BlockSpec((DE, 2 * D), lambda i: (0, 0)),
            pl.BlockSpec((1, 2 * D), lambda i: (0, 0)),
            pl.BlockSpec((BE, D), lambda i: (i, 0)),
            pl.BlockSpec((BE, D), lambda i: (i, 0)),
            pl.BlockSpec((BE, DE), lambda i: (i, 0)),
        ],
        out_specs=pl.BlockSpec((BE, D), lambda i: (i, 0)),
        out_shape=jax.ShapeDtypeStruct((E, D), jnp.float32),
    )(w1st, w2st, west, t, g1, g2, ef)


def _efstat_body(ef_ref, c16_ref, cs_ref):
    @pl.when(pl.program_id(0) == 0)
    def _init():
        c16_ref[...] = jnp.zeros_like(c16_ref)
        cs_ref[...] = jnp.zeros_like(cs_ref)

    ef = ef_ref[...]
    c16_ref[...] += jnp.dot(ef.T, ef, preferred_element_type=jnp.float32)
    cs_ref[...] += jnp.sum(ef, axis=0, keepdims=True)


def _efstats(ef):
    return pl.pallas_call(
        _efstat_body,
        grid=(NEB,),
        in_specs=[pl.BlockSpec((BE, DE), lambda i: (i, 0))],
        out_specs=[pl.BlockSpec((DE, DE), lambda i: (0, 0)),
                   pl.BlockSpec((1, DE), lambda i: (0, 0))],
        out_shape=[jax.ShapeDtypeStruct((DE, DE), jnp.float32),
                   jax.ShapeDtypeStruct((1, DE), jnp.float32)],
    )(ef)


def _stats_body(x_ref, s_ref, cnt1_ref, cnt2_ref, f1_ref, f2_ref, c16_ref,
                cs_ref, wt_ref, b_ref, g_ref, bb_ref,
                w1st_ref, w2st_ref, west_ref, t_ref):
    x = x_ref[...]
    w1t = wt_ref[:D, :]
    w2t = wt_ref[D:2 * D, :]
    wet = wt_ref[2 * D:, :]
    b = b_ref[...]
    y1 = jnp.dot(x, w1t, preferred_element_type=jnp.float32) + b
    y2 = jnp.dot(x, w2t, preferred_element_type=jnp.float32)
    t2 = jnp.dot(s_ref[...], w2t, preferred_element_type=jnp.float32)
    cnt1 = cnt1_ref[...]
    cnt2 = cnt2_ref[...]
    inv_e = 1.0 / E
    ea = jnp.sum(cnt1 * y1, axis=0, keepdims=True) * inv_e
    eb = jnp.sum(cnt2 * y2, axis=0, keepdims=True) * inv_e
    ec = jnp.dot(cs_ref[...], wet, preferred_element_type=jnp.float32) * inv_e
    m = ea + eb + ec
    ea2 = jnp.sum(cnt1 * y1 * y1, axis=0, keepdims=True) * inv_e
    eb2 = jnp.sum(cnt2 * y2 * y2, axis=0, keepdims=True) * inv_e
    cwet = jnp.dot(c16_ref[...], wet, preferred_element_type=jnp.float32)
    ec2 = jnp.sum(wet * cwet, axis=0, keepdims=True) * inv_e
    eab = jnp.sum(y1 * t2, axis=0, keepdims=True) * inv_e
    f1w = jnp.dot(f1_ref[...], wet, preferred_element_type=jnp.float32)
    eac = jnp.sum(y1 * f1w, axis=0, keepdims=True) * inv_e
    f2w = jnp.dot(f2_ref[...], wet, preferred_element_type=jnp.float32)
    ebc = jnp.sum(y2 * f2w, axis=0, keepdims=True) * inv_e
    v = ea2 + eb2 + ec2 + 2.0 * (eab + eac + ebc) - m * m
    s = g_ref[...] / jnp.sqrt(v + EPS)
    t = bb_ref[...] - m * s
    w1st_ref[...] = w1t * s
    w2st_ref[...] = w2t * s
    west_ref[...] = wet * s
    t_ref[...] = t + b * s


def _layer_stats(x, s_seg, cnt1, cnt2, f1, f2, c16, cs, wt, b, g, bb):
    full = lambda shp: pl.BlockSpec(shp, lambda: tuple(0 for _ in shp))
    return pl.pallas_call(
        _stats_body,
        in_specs=[full((N, D)), full((N, D)), full((N, 1)), full((N, 1)),
                  full((N, DE)), full((N, DE)), full((DE, DE)), full((1, DE)),
                  full((2 * D + DE, 2 * D)), full((1, 2 * D)),
                  full((1, 2 * D)), full((1, 2 * D))],
        out_specs=[full((D, 2 * D)), full((D, 2 * D)), full((DE, 2 * D)),
                   full((1, 2 * D))],
        out_shape=[jax.ShapeDtypeStruct((D, 2 * D), jnp.float32),
                   jax.ShapeDtypeStruct((D, 2 * D), jnp.float32),
                   jax.ShapeDtypeStruct((DE, 2 * D), jnp.float32),
                   jax.ShapeDtypeStruct((1, 2 * D), jnp.float32)],
    )(x, s_seg, cnt1, cnt2, f1, f2, c16, cs, wt, b, g, bb)


def _bn2_body(aggs_ref, cnt1_ref, x_ref, g_ref, b_ref, out_ref):
    agg = aggs_ref[...] / jnp.maximum(cnt1_ref[...], 1.0)
    m = jnp.mean(agg, axis=0, keepdims=True)
    v = jnp.mean((agg - m) ** 2, axis=0, keepdims=True)
    agg = g_ref[...] * (agg - m) / jnp.sqrt(v + EPS) + b_ref[...]
    out_ref[...] = _softplus(x_ref[...] + agg)


def _bn2_update(aggs, cnt1, x, g, b):
    full = lambda shp: pl.BlockSpec(shp, lambda: tuple(0 for _ in shp))
    return pl.pallas_call(
        _bn2_body,
        in_specs=[full((N, D)), full((N, 1)), full((N, D)), full((1, D)),
                  full((1, D))],
        out_specs=full((N, D)),
        out_shape=jax.ShapeDtypeStruct((N, D), jnp.float32),
    )(aggs, cnt1, x, g, b)


def _final_body(x_ref, idx3_ref, fc1_wt_ref, fc1_b_ref, out_wt_ref,
                out_b_ref, out_ref):
    x = x_ref[...]
    idx3 = idx3_ref[...]
    lanes = jax.lax.broadcasted_iota(jnp.int32, (N, 128), 1)
    onehot = (lanes == idx3).astype(jnp.float32)
    csum = jnp.dot(onehot.T, x, preferred_element_type=jnp.float32)
    ccnt = jnp.sum(onehot, axis=0, keepdims=True)
    crys = csum / jnp.maximum(ccnt.T, 1.0)
    crys = _softplus(
        jnp.dot(crys, fc1_wt_ref[...], preferred_element_type=jnp.float32)
        + fc1_b_ref[...])
    out = jnp.dot(crys, out_wt_ref[...], preferred_element_type=jnp.float32)
    out = out + out_b_ref[...]
    out_ref[...] = out[:G, :]


def _final(x, idx3, fc1_wt, fc1_b, out_wt, out_b):
    full = lambda shp: pl.BlockSpec(shp, lambda: tuple(0 for _ in shp))
    return pl.pallas_call(
        _final_body,
        in_specs=[full((N, D)), full((N, 1)), full((D, D)), full((1, D)),
                  full((D, 2)), full((1, 2))],
        out_specs=full((G, 2)),
        out_shape=jax.ShapeDtypeStruct((G, 2), jnp.float32),
    )(x, idx3, fc1_wt, fc1_b, out_wt, out_b)


def kernel(node_fea, edge_fea, idx1, idx2, idx3, emb, fc_full_W, fc_full_b,
           bn1_g, bn1_b, bn2_g, bn2_b, fc1_W, fc1_b, out_W, out_b):
    ef = edge_fea
    idx1 = idx1.astype(jnp.int32)
    idx2 = idx2.astype(jnp.int32)

    x = jnp.take(emb, node_fea, axis=0)

    ones_e = jnp.ones((E,), jnp.float32)
    cnt1 = jax.ops.segment_sum(ones_e, idx1, num_segments=N)[:, None]
    cnt2 = jax.ops.segment_sum(ones_e, idx2, num_segments=N)[:, None]
    f1 = jax.ops.segment_sum(ef, idx1, num_segments=N)
    f2 = jax.ops.segment_sum(ef, idx2, num_segments=N)
    c16, cs = _efstats(ef)

    for i in range(NC):
        wt = fc_full_W[i].T
        b = fc_full_b[i][None, :]
        g2 = jnp.take(x, idx2, axis=0)
        s_seg = jax.ops.segment_sum(g2, idx1, num_segments=N)
        w1st, w2st, west, t = _layer_stats(
            x, s_seg, cnt1, cnt2, f1, f2, c16, cs, wt, b,
            bn1_g[i][None, :], bn1_b[i][None, :])
        g1 = jnp.take(x, idx1, axis=0)
        w = _edge_stream(w1st, w2st, west, t, g1, g2, ef)
        aggs = jax.ops.segment_sum(w, idx1, num_segments=N)
        x = _bn2_update(aggs, cnt1, x, bn2_g[i][None, :], bn2_b[i][None, :])

    return _final(x, idx3.astype(jnp.int32)[:, None], fc1_W.T,
                  fc1_b[None, :], out_W.T, out_b[None, :])


# speedup vs baseline: 1.7119x; 1.7119x over previous
import functools

import jax
import jax.numpy as jnp
from jax import lax
from jax.experimental import pallas as pl
from jax.experimental.pallas import tpu as pltpu
from jax.experimental.pallas import tpu_sc as plsc

N = 10000
E = 320000
D = 128
DE = 16
NC = 3
G = 100
EPS = 1e-5
BE = 3200
NEB = E // BE

NCORE = 2
NSUB = 16
NW = NCORE * NSUB
CH = 80
EPW = E // NW
NCH = EPW // CH
NPAD = 10240
NPS = NPAD // NSUB


def _softplus(x):
    return jnp.maximum(x, 0.0) + jnp.log1p(jnp.exp(-jnp.abs(x)))


def _sigmoid(x):
    return 1.0 / (1.0 + jnp.exp(-x))


def _sc_mesh():
    return plsc.VectorSubcoreMesh(core_axis_name="c", subcore_axis_name="s")


@functools.lru_cache(maxsize=None)
def _build_sc_emb_gather():
    @functools.partial(
        pl.kernel,
        mesh=_sc_mesh(),
        out_type=jax.ShapeDtypeStruct((N, D), jnp.float32),
        scratch_types=[
            pltpu.VMEM((CH,), jnp.int32),
            pltpu.VMEM((CH, D), jnp.float32),
            pltpu.SemaphoreType.DMA,
        ],
    )
    def sc_emb_gather(emb_hbm, nf_hbm, x_hbm, nf_v, rows_v, sem):
        wid = lax.axis_index("c") * NSUB + lax.axis_index("s")

        @pl.when(wid < 25)
        def _():
            def body(c, carry):
                off = wid * 400 + c * CH
                pltpu.sync_copy(nf_hbm.at[pl.ds(off, CH)], nf_v)
                pltpu.async_copy(emb_hbm.at[nf_v], rows_v, sem).wait()
                pltpu.sync_copy(rows_v, x_hbm.at[pl.ds(off, CH)])
                return carry
            lax.fori_loop(0, 5, body, 0)

    return sc_emb_gather


@functools.lru_cache(maxsize=None)
def _build_sc_gather():
    @functools.partial(
        pl.kernel,
        mesh=_sc_mesh(),
        out_type=[jax.ShapeDtypeStruct((E, D), jnp.float32),
                  jax.ShapeDtypeStruct((E, D), jnp.float32)],
        scratch_types=[
            pltpu.VMEM((CH,), jnp.int32),
            pltpu.VMEM((CH,), jnp.int32),
            pltpu.VMEM((CH, D), jnp.float32),
            pltpu.VMEM((CH, D), jnp.float32),
            pltpu.SemaphoreType.DMA,
            pltpu.SemaphoreType.DMA,
        ],
    )
    def sc_gather(x_hbm, i1_hbm, i2_hbm,
                  g1_hbm, g2_hbm,
                  i1_v, i2_v, rows1_v, rows2_v, sem1, sem2):
        cid = lax.axis_index("c")
        sid = lax.axis_index("s")
        wid = cid * NSUB + sid

        def body(c, carry):
            off = wid * EPW + c * CH
            pltpu.sync_copy(i2_hbm.at[pl.ds(off, CH)], i2_v)
            pltpu.async_copy(x_hbm.at[i2_v], rows2_v, sem2).wait()
            pltpu.sync_copy(rows2_v, g2_hbm.at[pl.ds(off, CH)])
            pltpu.sync_copy(i1_hbm.at[pl.ds(off, CH)], i1_v)
            pltpu.async_copy(x_hbm.at[i1_v], rows1_v, sem1).wait()
            pltpu.sync_copy(rows1_v, g1_hbm.at[pl.ds(off, CH)])
            return carry
        lax.fori_loop(0, NCH, body, 0)

    return sc_gather


EPC = E // NCORE
SLAB = 5000
NSLAB = EPC // SLAB
TRASH = SLAB


@functools.lru_cache(maxsize=None)
def _build_sc_scatter():
    @functools.partial(
        pl.kernel,
        mesh=_sc_mesh(),
        out_type=[jax.ShapeDtypeStruct((NCORE * NPAD, D), jnp.float32),
                  jax.ShapeDtypeStruct((NCORE * NPAD, DE), jnp.float32)],
        scratch_types=[
            pltpu.VMEM((SLAB,), jnp.int32),
            pltpu.VMEM((SLAB + NSUB,), jnp.int32),
            pltpu.VMEM((SLAB + NSUB,), jnp.int32),
            pltpu.VMEM((CH, D), jnp.float32),
            pltpu.VMEM((NPS, D), jnp.float32),
            pltpu.VMEM((NPS, DE), jnp.float32),
            pltpu.SemaphoreType.DMA,
        ],
    )
    def sc_scatter(w_hbm, i1_hbm, zd_hbm, z16_hbm, agg_hbm, cnt_hbm,
                   ix_v, eid_v, lid_v, rows_v, acc_v, cacc_v, sem):
        cid = lax.axis_index("c")
        sid = lax.axis_index("s")
        rz = sid * NPS
        ebase = cid * EPC
        pltpu.sync_copy(zd_hbm.at[pl.ds(rz, NPS)], acc_v)
        pltpu.sync_copy(z16_hbm.at[pl.ds(rz, NPS)], cacc_v)
        one16 = jnp.ones((NSUB,), jnp.float32)
        lanes = jax.lax.broadcasted_iota(jnp.int32, (NSUB,), 0)
        zero16 = jnp.zeros((NSUB,), jnp.int32)

        def zinit(q, carry):
            eid_v[pl.ds(q * NSUB, NSUB)] = zero16
            lid_v[pl.ds(q * NSUB, NSUB)] = zero16
            return carry
        lax.fori_loop(0, (SLAB + NSUB) // NSUB, zinit, 0)

        def slab(sb, carry):
            sbase = ebase + sb * SLAB
            pltpu.sync_copy(i1_hbm.at[pl.ds(sbase, SLAB)], ix_v)

            def group(g, off):
                v = ix_v[pl.ds(g * NSUB, NSUB)]
                local = v - rz
                mask = (local >= 0) & (local < NPS)
                eids = (sbase + g * NSUB) + lanes
                csum = plsc.cumsum(mask.astype(jnp.int32))
                pos = jnp.where(mask, off + csum - 1, TRASH + lanes)
                plsc.store_scatter(eid_v, [pos], eids)
                plsc.store_scatter(lid_v, [pos], local)
                return off + csum[NSUB - 1]
            off = lax.fori_loop(0, SLAB // NSUB, group, 0)

            def chunk(c, carry):
                pltpu.async_copy(w_hbm.at[eid_v.at[pl.ds(c * CH, CH)]],
                                 rows_v, sem).wait()
                lo = c * CH
                hi = jnp.minimum(off - lo, CH)

                def acc_row(r, carry2):
                    lr = lid_v[pl.ds(lo + r, NSUB)][0]
                    for k in range(D // NSUB):
                        sl = pl.ds(k * NSUB, NSUB)
                        acc_v[lr, sl] = acc_v[lr, sl] + rows_v[r, sl]
                    cacc_v[lr, :] = cacc_v[lr, :] + one16
                    return carry2
                lax.fori_loop(0, hi, acc_row, 0)
                return carry
            lax.fori_loop(0, (off + CH - 1) // CH, chunk, 0)
            return carry

        lax.fori_loop(0, NSLAB, slab, 0)
        pltpu.sync_copy(acc_v, agg_hbm.at[pl.ds(cid * NPAD + rz, NPS)])
        pltpu.sync_copy(cacc_v, cnt_hbm.at[pl.ds(cid * NPAD + rz, NPS)])

    return sc_scatter


def _edge_body(w1st_ref, w2st_ref, west_ref, t_ref, g1_ref, g2_ref, ef_ref,
               w_ref):
    z = jnp.dot(g1_ref[...], w1st_ref[...], preferred_element_type=jnp.float32)
    z += jnp.dot(g2_ref[...], w2st_ref[...], preferred_element_type=jnp.float32)
    z += jnp.dot(ef_ref[...], west_ref[...], preferred_element_type=jnp.float32)
    z += t_ref[...]
    gate = _sigmoid(z[:, :D])
    conv = _softplus(z[:, D:])
    w_ref[...] = gate * conv


def _edge_stream(w1st, w2st, west, t, g1, g2, ef):
    return pl.pallas_call(
        _edge_body,
        grid=(NEB,),
        in_specs=[
            pl.BlockSpec((D, 2 * D), lambda i: (0, 0)),
            pl.BlockSpec((D, 2 * D), lambda i: (0, 0)),
            pl.BlockSpec((DE, 2 * D), lambda i: (0, 0)),
            pl.BlockSpec((1, 2 * D), lambda i: (0, 0)),
            pl.BlockSpec((BE, D), lambda i: (i, 0)),
            pl.BlockSpec((BE, D), lambda i: (i, 0)),
            pl.BlockSpec((BE, DE), lambda i: (i, 0)),
        ],
        out_specs=pl.BlockSpec((BE, D), lambda i: (i, 0)),
        out_shape=jax.ShapeDtypeStruct((E, D), jnp.float32),
    )(w1st, w2st, west, t, g1, g2, ef)


def _corr_body(g1_ref, g2_ref, ef_ref, mt_ref, m11_ref, m22_ref, kt_ref,
               lt_ref, s1_ref, s2_ref):
    @pl.when(pl.program_id(0) == 0)
    def _init():
        mt_ref[...] = jnp.zeros_like(mt_ref)
        m11_ref[...] = jnp.zeros_like(m11_ref)
        m22_ref[...] = jnp.zeros_like(m22_ref)
        kt_ref[...] = jnp.zeros_like(kt_ref)
        lt_ref[...] = jnp.zeros_like(lt_ref)
        s1_ref[...] = jnp.zeros_like(s1_ref)
        s2_ref[...] = jnp.zeros_like(s2_ref)

    g1 = g1_ref[...]
    g2 = g2_ref[...]
    ef = ef_ref[...]
    mt_ref[...] += jnp.dot(g2.T, g1, preferred_element_type=jnp.float32)
    m11_ref[...] += jnp.dot(g1.T, g1, preferred_element_type=jnp.float32)
    m22_ref[...] += jnp.dot(g2.T, g2, preferred_element_type=jnp.float32)
    kt_ref[...] += jnp.dot(ef.T, g1, preferred_element_type=jnp.float32)
    lt_ref[...] += jnp.dot(ef.T, g2, preferred_element_type=jnp.float32)
    s1_ref[...] += jnp.sum(g1, axis=0, keepdims=True)
    s2_ref[...] += jnp.sum(g2, axis=0, keepdims=True)


def _corr(g1, g2, ef):
    zero2 = lambda i: (0, 0)
    return pl.pallas_call(
        _corr_body,
        grid=(NEB,),
        in_specs=[pl.BlockSpec((BE, D), lambda i: (i, 0)),
                  pl.BlockSpec((BE, D), lambda i: (i, 0)),
                  pl.BlockSpec((BE, DE), lambda i: (i, 0))],
        out_specs=[pl.BlockSpec((D, D), zero2), pl.BlockSpec((D, D), zero2),
                   pl.BlockSpec((D, D), zero2), pl.BlockSpec((DE, D), zero2),
                   pl.BlockSpec((DE, D), zero2), pl.BlockSpec((1, D), zero2),
                   pl.BlockSpec((1, D), zero2)],
        out_shape=[jax.ShapeDtypeStruct((D, D), jnp.float32),
                   jax.ShapeDtypeStruct((D, D), jnp.float32),
                   jax.ShapeDtypeStruct((D, D), jnp.float32),
                   jax.ShapeDtypeStruct((DE, D), jnp.float32),
                   jax.ShapeDtypeStruct((DE, D), jnp.float32),
                   jax.ShapeDtypeStruct((1, D), jnp.float32),
                   jax.ShapeDtypeStruct((1, D), jnp.float32)],
    )(g1, g2, ef)


def _efstat_body(ef_ref, c16_ref, cs_ref):
    @pl.when(pl.program_id(0) == 0)
    def _init():
        c16_ref[...] = jnp.zeros_like(c16_ref)
        cs_ref[...] = jnp.zeros_like(cs_ref)

    ef = ef_ref[...]
    c16_ref[...] += jnp.dot(ef.T, ef, preferred_element_type=jnp.float32)
    cs_ref[...] += jnp.sum(ef, axis=0, keepdims=True)


def _efstats(ef):
    return pl.pallas_call(
        _efstat_body,
        grid=(NEB,),
        in_specs=[pl.BlockSpec((BE, DE), lambda i: (i, 0))],
        out_specs=[pl.BlockSpec((DE, DE), lambda i: (0, 0)),
                   pl.BlockSpec((1, DE), lambda i: (0, 0))],
        out_shape=[jax.ShapeDtypeStruct((DE, DE), jnp.float32),
                   jax.ShapeDtypeStruct((1, DE), jnp.float32)],
    )(ef)


def _stats_body(mt_ref, m11_ref, m22_ref, kt_ref, lt_ref, s1_ref, s2_ref,
                c16_ref, cs_ref, wt_ref, b_ref, g_ref, bb_ref,
                w1st_ref, w2st_ref, west_ref, t_ref):
    w1t = wt_ref[:D, :]
    w2t = wt_ref[D:2 * D, :]
    wet = wt_ref[2 * D:, :]
    b = b_ref[...]
    inv_e = 1.0 / E
    s1w = jnp.dot(s1_ref[...], w1t, preferred_element_type=jnp.float32)
    s2w = jnp.dot(s2_ref[...], w2t, preferred_element_type=jnp.float32)
    ea = s1w * inv_e + b
    eb = s2w * inv_e
    ec = jnp.dot(cs_ref[...], wet, preferred_element_type=jnp.float32) * inv_e
    m = ea + eb + ec
    m11w = jnp.dot(m11_ref[...], w1t, preferred_element_type=jnp.float32)
    ea2 = (jnp.sum(m11w * w1t, axis=0, keepdims=True) * inv_e
           + 2.0 * b * s1w * inv_e + b * b)
    m22w = jnp.dot(m22_ref[...], w2t, preferred_element_type=jnp.float32)
    eb2 = jnp.sum(m22w * w2t, axis=0, keepdims=True) * inv_e
    cwet = jnp.dot(c16_ref[...], wet, preferred_element_type=jnp.float32)
    ec2 = jnp.sum(wet * cwet, axis=0, keepdims=True) * inv_e
    mw1 = jnp.dot(mt_ref[...], w1t, preferred_element_type=jnp.float32)
    eab = (jnp.sum(mw1 * w2t, axis=0, keepdims=True) * inv_e + b * eb)
    kw1 = jnp.dot(kt_ref[...], w1t, preferred_element_type=jnp.float32)
    eac = jnp.sum(kw1 * wet, axis=0, keepdims=True) * inv_e + b * ec
    lw2 = jnp.dot(lt_ref[...], w2t, preferred_element_type=jnp.float32)
    ebc = jnp.sum(lw2 * wet, axis=0, keepdims=True) * inv_e
    v = ea2 + eb2 + ec2 + 2.0 * (eab + eac + ebc) - m * m
    s = g_ref[...] / jnp.sqrt(v + EPS)
    t = bb_ref[...] - m * s
    w1st_ref[...] = w1t * s
    w2st_ref[...] = w2t * s
    west_ref[...] = wet * s
    t_ref[...] = t + b * s


def _layer_stats(mt, m11, m22, kt, lt, s1, s2, c16, cs, wt, b, g, bb):
    full = lambda shp: pl.BlockSpec(shp, lambda: tuple(0 for _ in shp))
    return pl.pallas_call(
        _stats_body,
        in_specs=[full((D, D)), full((D, D)), full((D, D)),
                  full((DE, D)), full((DE, D)), full((1, D)), full((1, D)),
                  full((DE, DE)), full((1, DE)),
                  full((2 * D + DE, 2 * D)), full((1, 2 * D)),
                  full((1, 2 * D)), full((1, 2 * D))],
        out_specs=[full((D, 2 * D)), full((D, 2 * D)), full((DE, 2 * D)),
                   full((1, 2 * D))],
        out_shape=[jax.ShapeDtypeStruct((D, 2 * D), jnp.float32),
                   jax.ShapeDtypeStruct((D, 2 * D), jnp.float32),
                   jax.ShapeDtypeStruct((DE, 2 * D), jnp.float32),
                   jax.ShapeDtypeStruct((1, 2 * D), jnp.float32)],
    )(mt, m11, m22, kt, lt, s1, s2, c16, cs, wt, b, g, bb)


def _bn2_body(aggs_ref, cnt1_ref, x_ref, g_ref, b_ref, out_ref):
    agg = aggs_ref[...] / jnp.maximum(cnt1_ref[...], 1.0)
    m = jnp.mean(agg, axis=0, keepdims=True)
    v = jnp.mean((agg - m) ** 2, axis=0, keepdims=True)
    agg = g_ref[...] * (agg - m) / jnp.sqrt(v + EPS) + b_ref[...]
    out_ref[...] = _softplus(x_ref[...] + agg)


def _bn2_update(aggs, cnt1, x, g, b):
    full = lambda shp: pl.BlockSpec(shp, lambda: tuple(0 for _ in shp))
    return pl.pallas_call(
        _bn2_body,
        in_specs=[full((N, D)), full((N, 1)), full((N, D)),
                  full((1, D)), full((1, D))],
        out_specs=full((N, D)),
        out_shape=jax.ShapeDtypeStruct((N, D), jnp.float32),
    )(aggs, cnt1, x, g, b)


def _final_body(x_ref, idx3_ref, fc1_wt_ref, fc1_b_ref, out_wt_ref,
                out_b_ref, out_ref):
    x = x_ref[...]
    idx3 = idx3_ref[...]
    lanes = jax.lax.broadcasted_iota(jnp.int32, (N, 128), 1)
    onehot = (lanes == idx3).astype(jnp.float32)
    csum = jnp.dot(onehot.T, x, preferred_element_type=jnp.float32)
    ccnt = jnp.sum(onehot, axis=0, keepdims=True)
    crys = csum / jnp.maximum(ccnt.T, 1.0)
    crys = _softplus(
        jnp.dot(crys, fc1_wt_ref[...], preferred_element_type=jnp.float32)
        + fc1_b_ref[...])
    out = jnp.dot(crys, out_wt_ref[...], preferred_element_type=jnp.float32)
    out = out + out_b_ref[...]
    out_ref[...] = out[:G, :]


def _final(x, idx3, fc1_wt, fc1_b, out_wt, out_b):
    full = lambda shp: pl.BlockSpec(shp, lambda: tuple(0 for _ in shp))
    return pl.pallas_call(
        _final_body,
        in_specs=[full((N, D)), full((N, 1)), full((D, D)), full((1, D)),
                  full((D, 2)), full((1, 2))],
        out_specs=full((G, 2)),
        out_shape=jax.ShapeDtypeStruct((G, 2), jnp.float32),
    )(x, idx3, fc1_wt, fc1_b, out_wt, out_b)


def kernel(node_fea, edge_fea, idx1, idx2, idx3, emb, fc_full_W, fc_full_b,
           bn1_g, bn1_b, bn2_g, bn2_b, fc1_W, fc1_b, out_W, out_b):
    ef = edge_fea
    i1 = idx1.astype(jnp.int32)
    i2 = idx2.astype(jnp.int32)
    nf = node_fea.astype(jnp.int32)
    zd = jnp.zeros((NPAD, D), jnp.float32)
    z16 = jnp.zeros((NPAD, DE), jnp.float32)

    x0 = _build_sc_emb_gather()(emb, nf)
    c16, cs = _efstats(ef)

    def layer_step(x, p):
        wt, b, g1n, b1n, g2n, b2n = p
        g1, g2 = _build_sc_gather()(x, i1, i2)
        mt, m11, m22, kt, lt, s1, s2 = _corr(g1, g2, ef)
        w1st, w2st, west, t = _layer_stats(
            mt, m11, m22, kt, lt, s1, s2, c16, cs, wt, b, g1n, b1n)
        w = _edge_stream(w1st, w2st, west, t, g1, g2, ef)
        aggsum = jax.ops.segment_sum(w, i1, num_segments=N)
        cnt1 = jax.ops.segment_sum(jnp.ones((E, 1), jnp.float32), i1,
                                   num_segments=N)
        return _bn2_update(aggsum, cnt1, x, g2n, b2n), None

    params = (jnp.swapaxes(fc_full_W, 1, 2), fc_full_b[:, None, :],
              bn1_g[:, None, :], bn1_b[:, None, :],
              bn2_g[:, None, :], bn2_b[:, None, :])
    x, _ = lax.scan(layer_step, x0, params)

    return _final(x, idx3.astype(jnp.int32)[:, None], fc1_W.T,
                  fc1_b[None, :], out_W.T, out_b[None, :])
